# baseline (speedup 1.0000x reference)
import jax
import jax.numpy as jnp
from jax import lax
from jax.experimental import pallas as pl
from jax.experimental.pallas import tpu as pltpu
from jax.experimental.pallas import tpu_sc as plsc

HASH_VOCAB = 100000
EMBED_DIM = 128
VOCAB = 260
NGRAMS = (3, 4, 5, 6, 7, 8)
BASE = 257
BATCH = 32
SEQ = 2048
LANES = 16
PAD = 8
CHUNK = 32
NCHUNK = SEQ // CHUNK
NTAB = len(NGRAMS)
NBUF = 8
NVEC = EMBED_DIM // LANES
NGRP = CHUNK // LANES
INV_N = 1.0 / float(NTAB)


def _mod_hash_vocab(x):
  q = (x.astype(jnp.float32) * (1.0 / HASH_VOCAB)).astype(jnp.int32)
  r = x - q * HASH_VOCAB
  r = jnp.where(r < 0, r + HASH_VOCAB, r)
  r = jnp.where(r >= HASH_VOCAB, r - HASH_VOCAB, r)
  return r


def _body(ids_hbm, byte_hbm, t3, t4, t5, t6, t7, t8, out_hbm,
          bytes_v, *rest):
  idxs = rest[:NBUF]
  byte_bufs = rest[NBUF:2 * NBUF]
  acc_bufs = rest[2 * NBUF:3 * NBUF]
  sems = rest[3 * NBUF:]
  c = lax.axis_index("c")
  s = lax.axis_index("s")
  row = s * 2 + c

  bytes_v[pl.ds(0, LANES)] = jnp.zeros((LANES,), jnp.int32)
  pltpu.sync_copy(ids_hbm.at[pl.ds(row * SEQ, SEQ)], bytes_v.at[pl.ds(PAD, SEQ)])

  zeros = jnp.zeros((LANES,), jnp.float32)

  def zero_all(i, _):
    t = i // NVEC
    d = i % NVEC
    for b in range(NBUF):
      acc_bufs[b][t, pl.ds(d * LANES, LANES)] = zeros
    return 0

  lax.fori_loop(0, CHUNK * NVEC, zero_all, 0)

  tabs = (t3, t4, t5, t6, t7, t8)

  def fire(b, ch):
    s0 = ch * CHUNK
    for g in range(NGRP):
      p = s0 + g * LANES
      b0 = bytes_v[pl.ds(PAD + p, LANES)]
      pos = p + lax.iota(jnp.int32, LANES)
      h = b0
      for j in range(1, NGRAMS[-1]):
        bj = bytes_v[pl.ds(PAD + p - j, LANES)]
        h = _mod_hash_vocab(h * BASE + bj)
        n = j + 1
        if n >= NGRAMS[0]:
          k = n - NGRAMS[0]
          idxs[b][k, pl.ds(g * LANES, LANES)] = jnp.where(pos < n - 1, b0, h)
    pltpu.async_copy(
        byte_hbm.at[bytes_v.at[pl.ds(PAD + s0, CHUNK)]], byte_bufs[b], sems[b])
    for k in range(NTAB):
      pltpu.async_copy(tabs[k].at[idxs[b].at[k]], acc_bufs[b], sems[b],
                       add=True)

  def drain(b, ch):
    s0 = ch * CHUNK
    pltpu.make_async_copy(
        byte_hbm.at[bytes_v.at[pl.ds(PAD + s0, CHUNK)]], byte_bufs[b],
        sems[b]).wait()
    for k in range(NTAB):
      pltpu.make_async_copy(
          tabs[k].at[idxs[b].at[k]], acc_bufs[b], sems[b]).wait()

  def compute_store(b, ch):
    g0 = byte_bufs[b]
    av = acc_bufs[b]

    def tok_body(i, _):
      for d in range(NVEC):
        sl = pl.ds(d * LANES, LANES)
        g0[i, sl] = g0[i, sl] + av[i, sl] * INV_N
        av[i, sl] = zeros
      return 0

    lax.fori_loop(0, CHUNK, tok_body, 0)
    pltpu.sync_copy(g0, out_hbm.at[row, pl.ds(ch * CHUNK, CHUNK)])

  for b in range(NBUF):
    fire(b, b)

  def round_body(i, _):
    for b in range(NBUF):
      ch = i * NBUF + b
      drain(b, ch)
      compute_store(b, ch)
      nxt = ch + NBUF

      @pl.when(nxt < NCHUNK)
      def _():
        fire(b, nxt)
    return 0

  lax.fori_loop(0, NCHUNK // NBUF, round_body, 0)


@jax.jit
def kernel(input_ids, byte_table, hash_table_3, hash_table_4, hash_table_5,
           hash_table_6, hash_table_7, hash_table_8):
  mesh = plsc.VectorSubcoreMesh(core_axis_name="c", subcore_axis_name="s")
  run = pl.kernel(
      _body,
      out_type=jax.ShapeDtypeStruct((BATCH, SEQ, EMBED_DIM), jnp.float32),
      mesh=mesh,
      scratch_types=[
          pltpu.VMEM((SEQ + PAD,), jnp.int32),
      ] + [pltpu.VMEM((NTAB, CHUNK), jnp.int32)] * NBUF
        + [pltpu.VMEM((CHUNK, EMBED_DIM), jnp.float32)] * (2 * NBUF) + [
          pltpu.SemaphoreType.DMA,
      ] * NBUF,
  )
  return run(input_ids.astype(jnp.int32).reshape(-1), byte_table, hash_table_3,
             hash_table_4, hash_table_5, hash_table_6, hash_table_7,
             hash_table_8)


# speedup vs baseline: 1.0320x; 1.0320x over previous
import jax
import jax.numpy as jnp
from jax import lax
from jax.experimental import pallas as pl
from jax.experimental.pallas import tpu as pltpu
from jax.experimental.pallas import tpu_sc as plsc

HASH_VOCAB = 100000
EMBED_DIM = 128
VOCAB = 260
NGRAMS = (3, 4, 5, 6, 7, 8)
BASE = 257
BATCH = 32
SEQ = 2048
LANES = 16
PAD = 8
CHUNK = 32
NCHUNK = SEQ // CHUNK
NTAB = len(NGRAMS)
NBUF = 4
NVEC = EMBED_DIM // LANES
NGRP = CHUNK // LANES
INV_N = 1.0 / float(NTAB)


def _mod_hash_vocab(x):
  q = (x.astype(jnp.float32) * (1.0 / HASH_VOCAB)).astype(jnp.int32)
  r = x - q * HASH_VOCAB
  r = jnp.where(r < 0, r + HASH_VOCAB, r)
  r = jnp.where(r >= HASH_VOCAB, r - HASH_VOCAB, r)
  return r


def _body(ids_hbm, byte_hbm, t3, t4, t5, t6, t7, t8, out_hbm,
          bytes_v, *rest):
  idxs = rest[:NBUF]
  bufs = [rest[NBUF + i * 7:NBUF + (i + 1) * 7] for i in range(NBUF)]
  sems = rest[NBUF + NBUF * 7:]
  c = lax.axis_index("c")
  s = lax.axis_index("s")
  row = s * 2 + c

  bytes_v[pl.ds(0, LANES)] = jnp.zeros((LANES,), jnp.int32)
  pltpu.sync_copy(ids_hbm.at[pl.ds(row * SEQ, SEQ)], bytes_v.at[pl.ds(PAD, SEQ)])

  tabs = (t3, t4, t5, t6, t7, t8)

  def fire(b, ch):
    s0 = ch * CHUNK
    for g in range(NGRP):
      p = s0 + g * LANES
      b0 = bytes_v[pl.ds(PAD + p, LANES)]
      pos = p + lax.iota(jnp.int32, LANES)
      h = b0
      for j in range(1, NGRAMS[-1]):
        bj = bytes_v[pl.ds(PAD + p - j, LANES)]
        h = _mod_hash_vocab(h * BASE + bj)
        n = j + 1
        if n >= NGRAMS[0]:
          k = n - NGRAMS[0]
          idxs[b][k, pl.ds(g * LANES, LANES)] = jnp.where(pos < n - 1, b0, h)
    pltpu.async_copy(
        byte_hbm.at[bytes_v.at[pl.ds(PAD + s0, CHUNK)]], bufs[b][0], sems[b])
    for k in range(NTAB):
      pltpu.async_copy(tabs[k].at[idxs[b].at[k]], bufs[b][k + 1], sems[b])

  def drain(b, ch):
    s0 = ch * CHUNK
    pltpu.make_async_copy(
        byte_hbm.at[bytes_v.at[pl.ds(PAD + s0, CHUNK)]], bufs[b][0],
        sems[b]).wait()
    for k in range(NTAB):
      pltpu.make_async_copy(
          tabs[k].at[idxs[b].at[k]], bufs[b][k + 1], sems[b]).wait()

  def compute_store(b, ch):
    g0, g1, g2, g3, g4, g5, g6 = bufs[b]

    def tok_body(i, _):
      for d in range(NVEC):
        sl = pl.ds(d * LANES, LANES)
        acc = g1[i, sl] + g2[i, sl]
        acc = acc + g3[i, sl]
        acc = acc + g4[i, sl]
        acc = acc + g5[i, sl]
        acc = acc + g6[i, sl]
        g0[i, sl] = g0[i, sl] + acc * INV_N
      return 0

    lax.fori_loop(0, CHUNK, tok_body, 0)
    pltpu.sync_copy(g0, out_hbm.at[row, pl.ds(ch * CHUNK, CHUNK)])

  for b in range(NBUF):
    fire(b, b)

  def round_body(i, _):
    for b in range(NBUF):
      ch = i * NBUF + b
      drain(b, ch)
      compute_store(b, ch)
      nxt = ch + NBUF

      @pl.when(nxt < NCHUNK)
      def _():
        fire(b, nxt)
    return 0

  lax.fori_loop(0, NCHUNK // NBUF, round_body, 0)


@jax.jit
def kernel(input_ids, byte_table, hash_table_3, hash_table_4, hash_table_5,
           hash_table_6, hash_table_7, hash_table_8):
  mesh = plsc.VectorSubcoreMesh(core_axis_name="c", subcore_axis_name="s")
  run = pl.kernel(
      _body,
      out_type=jax.ShapeDtypeStruct((BATCH, SEQ, EMBED_DIM), jnp.float32),
      mesh=mesh,
      scratch_types=[
          pltpu.VMEM((SEQ + PAD,), jnp.int32),
      ] + [pltpu.VMEM((NTAB, CHUNK), jnp.int32)] * NBUF
        + [pltpu.VMEM((CHUNK, EMBED_DIM), jnp.float32)] * (7 * NBUF) + [
          pltpu.SemaphoreType.DMA,
      ] * NBUF,
  )
  return run(input_ids.astype(jnp.int32).reshape(-1), byte_table, hash_table_3,
             hash_table_4, hash_table_5, hash_table_6, hash_table_7,
             hash_table_8)
